# baseline (speedup 1.0000x reference)
import functools

import jax
import jax.numpy as jnp
from jax import lax
from jax.experimental import pallas as pl
from jax.experimental.pallas import tpu as pltpu
from jax.experimental.pallas import tpu_sc as plsc

D_MODEL = 1024
SEQ_LEN = 8192
LANES = 16
VECS_PER_ROW = D_MODEL // LANES

NUM_CORES = 2
NUM_SUBCORES = 16
NUM_WORKERS = NUM_CORES * NUM_SUBCORES
ROWS_PER_WORKER = SEQ_LEN // NUM_WORKERS
CHUNK = 16
NUM_CHUNKS = ROWS_PER_WORKER // CHUNK
NE = 3
NX = 4
AHEAD = 2

_mesh = plsc.VectorSubcoreMesh(core_axis_name="c", subcore_axis_name="s")


@functools.partial(
    pl.kernel,
    out_type=jax.ShapeDtypeStruct((SEQ_LEN, D_MODEL), jnp.float32),
    mesh=_mesh,
    scratch_types=[
        pltpu.VMEM((ROWS_PER_WORKER,), jnp.int32),
        [pltpu.VMEM((CHUNK, D_MODEL), jnp.float32) for _ in range(NE)],
        [pltpu.VMEM((CHUNK, D_MODEL), jnp.float32) for _ in range(NX)],
        [pltpu.SemaphoreType.DMA for _ in range(NE)],
        [pltpu.SemaphoreType.DMA for _ in range(NX)],
        [pltpu.SemaphoreType.DMA for _ in range(NX)],
    ],
)
def _pos_encode(x_hbm, idx_hbm, table_hbm, out_hbm,
                idx_v, ebufs, xbufs, gsems, xsems, osems):
    wid = lax.axis_index("s") * NUM_CORES + lax.axis_index("c")
    base = wid * ROWS_PER_WORKER

    def idx_vec(i):
        return idx_v[pl.ds(i * CHUNK, CHUNK)]

    def start_gather(i):
        pltpu.async_copy(table_hbm.at[idx_vec(i)], ebufs[i % NE], gsems[i % NE])

    def wait_gather(i):
        pltpu.make_async_copy(
            table_hbm.at[idx_vec(i)], ebufs[i % NE], gsems[i % NE]).wait()

    def start_x(i):
        pltpu.async_copy(x_hbm.at[pl.ds(base + i * CHUNK, CHUNK)],
                         xbufs[i % NX], xsems[i % NX])

    def wait_x(i):
        pltpu.make_async_copy(x_hbm.at[pl.ds(base + i * CHUNK, CHUNK)],
                              xbufs[i % NX], xsems[i % NX]).wait()

    def start_out(i):
        pltpu.async_copy(xbufs[i % NX],
                         out_hbm.at[pl.ds(base + i * CHUNK, CHUNK)],
                         osems[i % NX])

    def wait_out(i):
        pltpu.make_async_copy(xbufs[i % NX],
                              out_hbm.at[pl.ds(base + i * CHUNK, CHUNK)],
                              osems[i % NX]).wait()

    pltpu.sync_copy(idx_hbm.at[pl.ds(base, ROWS_PER_WORKER)], idx_v)
    for j in range(NE):
        start_gather(j)
    for j in range(AHEAD):
        start_x(j)

    for i in range(NUM_CHUNKS):
        wait_gather(i)
        wait_x(i)
        if i + AHEAD < NUM_CHUNKS:
            if i + AHEAD >= NX:
                wait_out(i + AHEAD - NX)
            start_x(i + AHEAD)
        eb, xb = ebufs[i % NE], xbufs[i % NX]

        def add_row(r, carry):
            for v in range(VECS_PER_ROW):
                c = v * LANES
                plsc.addupdate(xb.at[r, pl.ds(c, LANES)],
                               eb[r, pl.ds(c, LANES)])
            return carry

        lax.fori_loop(0, CHUNK, add_row, 0)
        if i + NE < NUM_CHUNKS:
            start_gather(i + NE)
        start_out(i)

    for i in range(NUM_CHUNKS - NX, NUM_CHUNKS):
        wait_out(i)


def kernel(x, indices, table):
    return _pos_encode(x, indices.astype(jnp.int32), table)


# speedup vs baseline: 1.0085x; 1.0085x over previous
import functools

import jax
import jax.numpy as jnp
from jax import lax
from jax.experimental import pallas as pl
from jax.experimental.pallas import tpu as pltpu
from jax.experimental.pallas import tpu_sc as plsc

D_MODEL = 1024
SEQ_LEN = 8192
LANES = 16
VECS_PER_ROW = D_MODEL // LANES

NUM_CORES = 2
NUM_SUBCORES = 16
NUM_WORKERS = NUM_CORES * NUM_SUBCORES
ROWS_PER_WORKER = SEQ_LEN // NUM_WORKERS
CHUNK = 16
NUM_CHUNKS = ROWS_PER_WORKER // CHUNK
NE = 3
NX = 4
AHEAD = 2

_mesh = plsc.VectorSubcoreMesh(core_axis_name="c", subcore_axis_name="s")


@functools.partial(
    pl.kernel,
    out_type=jax.ShapeDtypeStruct((SEQ_LEN, D_MODEL), jnp.float32),
    mesh=_mesh,
    scratch_types=[
        pltpu.VMEM((ROWS_PER_WORKER,), jnp.int32),
        [pltpu.VMEM((CHUNK, D_MODEL), jnp.float32) for _ in range(NE)],
        [pltpu.VMEM((CHUNK, D_MODEL), jnp.float32) for _ in range(NX)],
        [pltpu.SemaphoreType.DMA for _ in range(NE)],
        [pltpu.SemaphoreType.DMA for _ in range(NX)],
        [pltpu.SemaphoreType.DMA for _ in range(NX)],
    ],
)
def _pos_encode(x_hbm, idx_hbm, table_hbm, out_hbm,
                idx_v, ebufs, xbufs, gsems, xsems, osems):
    wid = lax.axis_index("c") * NUM_SUBCORES + lax.axis_index("s")
    base = wid * ROWS_PER_WORKER

    def idx_vec(i):
        return idx_v[pl.ds(i * CHUNK, CHUNK)]

    def start_gather(i):
        pltpu.async_copy(table_hbm.at[idx_vec(i)], ebufs[i % NE], gsems[i % NE])

    def wait_gather(i):
        pltpu.make_async_copy(
            table_hbm.at[idx_vec(i)], ebufs[i % NE], gsems[i % NE]).wait()

    def start_x(i):
        pltpu.async_copy(x_hbm.at[pl.ds(base + i * CHUNK, CHUNK)],
                         xbufs[i % NX], xsems[i % NX])

    def wait_x(i):
        pltpu.make_async_copy(x_hbm.at[pl.ds(base + i * CHUNK, CHUNK)],
                              xbufs[i % NX], xsems[i % NX]).wait()

    def start_out(i):
        pltpu.async_copy(xbufs[i % NX],
                         out_hbm.at[pl.ds(base + i * CHUNK, CHUNK)],
                         osems[i % NX])

    def wait_out(i):
        pltpu.make_async_copy(xbufs[i % NX],
                              out_hbm.at[pl.ds(base + i * CHUNK, CHUNK)],
                              osems[i % NX]).wait()

    pltpu.sync_copy(idx_hbm.at[pl.ds(base, ROWS_PER_WORKER)], idx_v)
    for j in range(AHEAD):
        start_gather(j)
        start_x(j)

    for i in range(NUM_CHUNKS):
        wait_gather(i)
        wait_x(i)
        if i + AHEAD < NUM_CHUNKS:
            if i + AHEAD >= NX:
                wait_out(i + AHEAD - NX)
            start_x(i + AHEAD)
            start_gather(i + AHEAD)
        eb, xb = ebufs[i % NE], xbufs[i % NX]

        def add_row(r, carry):
            for v in range(VECS_PER_ROW):
                c = v * LANES
                plsc.addupdate(xb.at[r, pl.ds(c, LANES)],
                               eb[r, pl.ds(c, LANES)])
            return carry

        lax.fori_loop(0, CHUNK, add_row, 0)
        start_out(i)

    for i in range(NUM_CHUNKS - NX, NUM_CHUNKS):
        wait_out(i)


def kernel(x, indices, table):
    return _pos_encode(x, indices.astype(jnp.int32), table)
